# Initial kernel scaffold; baseline (speedup 1.0000x reference)
import jax
import jax.numpy as jnp
from jax.experimental import pallas as pl


def kernel(x, W1_0, g1_0, b1_0, W2_0, g2_0, b2_0, W1_1, g1_1, b1_1, W2_1, g2_1, b2_1):
    raise NotImplementedError("write your pallas kernel here")


# speedup vs baseline: 2.6049x; 2.6049x over previous
import functools
import jax
import jax.numpy as jnp
from jax import lax
from jax.experimental import pallas as pl

EPS = 1e-5
B = 8
N = 1024
K = 32
TILE_N = 256
S_TILE = TILE_N * K
CNT = B * N * K


def _leaky(y):
    return jnp.where(y >= 0, y, 0.2 * y)


def _nd_body(tab_ref, o_ref):
    tb = tab_ref[0]
    G = lax.dot_general(tb, tb, (((0,), (0,)), ((), ())))
    xx = jnp.sum(tb * tb, axis=0)
    inner = -2.0 * G
    o_ref[0] = (-xx[:, None] - inner) - xx[None, :]


def _neg_dist(tab):
    C = tab.shape[1]
    return pl.pallas_call(
        _nd_body,
        grid=(B,),
        in_specs=[pl.BlockSpec((1, C, N), lambda b: (b, 0, 0))],
        out_specs=pl.BlockSpec((1, N, N), lambda b: (b, 0, 0)),
        out_shape=jax.ShapeDtypeStruct((B, N, N), jnp.float32),
    )(tab)


def _gather_lanes(tab, idxb):
    C, S = idxb.shape
    out = jnp.zeros((C, S), jnp.float32)
    for w in range(N // 128):
        lo = w * 128
        idx_loc = jnp.clip(idxb - lo, 0, 127)
        g = jnp.take_along_axis(tab[:, lo:lo + 128], idx_loc, axis=1)
        m = jnp.logical_and(idxb >= lo, idxb < lo + 128)
        out = jnp.where(m, g, out)
    return out


def _k3_body(tab_ref, idx_ref, w1_ref, y1_ref, st_ref, *, C):
    t = pl.program_id(1)
    tb = tab_ref[0]
    idx = idx_ref[0, 0]
    idxb = jnp.broadcast_to(idx[None, :], (C, S_TILE))
    feat = _gather_lanes(tb, idxb)
    base = t * TILE_N
    nidx = base + lax.broadcasted_iota(jnp.int32, (C, S_TILE), 1) // K
    center = _gather_lanes(tb, nidx)
    g = jnp.concatenate([center, feat - center], axis=0)
    y1 = lax.dot_general(w1_ref[...], g, (((1,), (0,)), ((), ())))
    y1_ref[0] = y1
    part = jnp.stack([jnp.sum(y1, axis=1), jnp.sum(y1 * y1, axis=1)], axis=0)

    @pl.when(jnp.logical_and(pl.program_id(0) == 0, t == 0))
    def _():
        st_ref[...] = jnp.zeros_like(st_ref)

    st_ref[...] += part


def _k3(tab, idx_r, W1):
    C = tab.shape[1]
    body = functools.partial(_k3_body, C=C)
    return pl.pallas_call(
        body,
        grid=(B, N // TILE_N),
        in_specs=[
            pl.BlockSpec((1, C, N), lambda b, t: (b, 0, 0)),
            pl.BlockSpec((1, 1, S_TILE), lambda b, t: (b * (N // TILE_N) + t, 0, 0)),
            pl.BlockSpec((64, 2 * C), lambda b, t: (0, 0)),
        ],
        out_specs=[
            pl.BlockSpec((1, 64, S_TILE), lambda b, t: (b, 0, t)),
            pl.BlockSpec((2, 64), lambda b, t: (0, 0)),
        ],
        out_shape=[
            jax.ShapeDtypeStruct((B, 64, N * K), jnp.float32),
            jax.ShapeDtypeStruct((2, 64), jnp.float32),
        ],
    )(tab, idx_r, W1)


def _k4_body(y1_ref, st_ref, g1_ref, b1_ref, w2_ref, a1_ref, st2_ref):
    s1 = st_ref[0]
    s2 = st_ref[1]
    mean = s1 / CNT
    var = s2 / CNT - mean * mean
    sc = g1_ref[0] / jnp.sqrt(var + EPS)
    bi = b1_ref[0] - sc * mean
    a1 = _leaky(sc[:, None] * y1_ref[0] + bi[:, None])
    a1_ref[0] = a1
    y2 = lax.dot_general(w2_ref[...], a1, (((1,), (0,)), ((), ())))
    part = jnp.stack([jnp.sum(y2, axis=1), jnp.sum(y2 * y2, axis=1)], axis=0)

    @pl.when(jnp.logical_and(pl.program_id(0) == 0, pl.program_id(1) == 0))
    def _():
        st2_ref[...] = jnp.zeros_like(st2_ref)

    st2_ref[...] += part


def _k4(y1, stats1, g1, b1, W2):
    return pl.pallas_call(
        _k4_body,
        grid=(B, N // TILE_N),
        in_specs=[
            pl.BlockSpec((1, 64, S_TILE), lambda b, t: (b, 0, t)),
            pl.BlockSpec((2, 64), lambda b, t: (0, 0)),
            pl.BlockSpec((1, 64), lambda b, t: (0, 0)),
            pl.BlockSpec((1, 64), lambda b, t: (0, 0)),
            pl.BlockSpec((64, 64), lambda b, t: (0, 0)),
        ],
        out_specs=[
            pl.BlockSpec((1, 64, S_TILE), lambda b, t: (b, 0, t)),
            pl.BlockSpec((2, 64), lambda b, t: (0, 0)),
        ],
        out_shape=[
            jax.ShapeDtypeStruct((B, 64, N * K), jnp.float32),
            jax.ShapeDtypeStruct((2, 64), jnp.float32),
        ],
    )(y1, stats1, g1.reshape(1, 64), b1.reshape(1, 64), W2)


def _k5_body(a1_ref, st2_ref, w2_ref, g2_ref, b2_ref, o_ref):
    W2 = w2_ref[...]
    mean = st2_ref[0] / CNT
    var = st2_ref[1] / CNT - mean * mean
    sc = g2_ref[0] / jnp.sqrt(var + EPS)
    bi = b2_ref[0] - sc * mean
    sgn = jnp.where(sc >= 0, 1.0, -1.0)
    W2s = sgn[:, None] * W2
    y2 = lax.dot_general(W2s, a1_ref[0], (((1,), (0,)), ((), ())))
    zh = jnp.max(y2.reshape(64, TILE_N, K), axis=-1)
    z = sgn[:, None] * zh
    o_ref[0] = _leaky(sc[:, None] * z + bi[:, None])


def _k5(a1, stats2, W2, g2, b2):
    return pl.pallas_call(
        _k5_body,
        grid=(B, N // TILE_N),
        in_specs=[
            pl.BlockSpec((1, 64, S_TILE), lambda b, t: (b, 0, t)),
            pl.BlockSpec((2, 64), lambda b, t: (0, 0)),
            pl.BlockSpec((64, 64), lambda b, t: (0, 0)),
            pl.BlockSpec((1, 64), lambda b, t: (0, 0)),
            pl.BlockSpec((1, 64), lambda b, t: (0, 0)),
        ],
        out_specs=pl.BlockSpec((1, 64, TILE_N), lambda b, t: (b, 0, t)),
        out_shape=jax.ShapeDtypeStruct((B, 64, N), jnp.float32),
    )(a1, stats2, W2, g2.reshape(1, 64), b2.reshape(1, 64))


def _topk_idx(nd):
    _, idx = jax.lax.top_k(nd, K)
    return idx


def _edge_layer(tab, W1, g1, b1, W2, g2, b2):
    nd = _neg_dist(tab)
    idx = _topk_idx(nd)
    idx_r = idx.reshape(B * (N // TILE_N), 1, S_TILE)
    y1, stats1 = _k3(tab, idx_r, W1)
    a1, stats2 = _k4(y1, stats1, g1, b1, W2)
    return _k5(a1, stats2, W2, g2, b2)


def kernel(x, W1_0, g1_0, b1_0, W2_0, g2_0, b2_0, W1_1, g1_1, b1_1, W2_1, g2_1, b2_1):
    h = _edge_layer(x, W1_0, g1_0, b1_0, W2_0, g2_0, b2_0)
    h = _edge_layer(h, W1_1, g1_1, b1_1, W2_1, g2_1, b2_1)
    return h


# speedup vs baseline: 4.7927x; 1.8399x over previous
import functools
import jax
import jax.numpy as jnp
from jax import lax
from jax.experimental import pallas as pl
from jax.experimental.pallas import tpu as pltpu
from jax.experimental.pallas import tpu_sc as plsc

EPS = 1e-5
B = 8
N = 1024
K = 32
TILE_N = 256
S_TILE = TILE_N * K
CNT = B * N * K


def _leaky(y):
    return jnp.where(y >= 0, y, 0.2 * y)


def _nd_body(tab_ref, o_ref):
    tb = tab_ref[0]
    G = lax.dot_general(tb, tb, (((0,), (0,)), ((), ())))
    xx = jnp.sum(tb * tb, axis=0)
    inner = -2.0 * G
    o_ref[0] = (-xx[:, None] - inner) - xx[None, :]


def _neg_dist(tab):
    C = tab.shape[1]
    return pl.pallas_call(
        _nd_body,
        grid=(B,),
        in_specs=[pl.BlockSpec((1, C, N), lambda b: (b, 0, 0))],
        out_specs=pl.BlockSpec((1, N, N), lambda b: (b, 0, 0)),
        out_shape=jax.ShapeDtypeStruct((B, N, N), jnp.float32),
    )(tab)


def _gather_lanes(tab, idxb):
    C, S = idxb.shape
    out = jnp.zeros((C, S), jnp.float32)
    for w in range(N // 128):
        lo = w * 128
        idx_loc = jnp.clip(idxb - lo, 0, 127)
        g = jnp.take_along_axis(tab[:, lo:lo + 128], idx_loc, axis=1)
        m = jnp.logical_and(idxb >= lo, idxb < lo + 128)
        out = jnp.where(m, g, out)
    return out


def _k3_body(tab_ref, idx_ref, w1_ref, y1_ref, st_ref, *, C):
    t = pl.program_id(1)
    tb = tab_ref[0]
    idx = idx_ref[0, 0]
    idxb = jnp.broadcast_to(idx[None, :], (C, S_TILE))
    feat = _gather_lanes(tb, idxb)
    base = t * TILE_N
    nidx = base + lax.broadcasted_iota(jnp.int32, (C, S_TILE), 1) // K
    center = _gather_lanes(tb, nidx)
    g = jnp.concatenate([center, feat - center], axis=0)
    y1 = lax.dot_general(w1_ref[...], g, (((1,), (0,)), ((), ())))
    y1_ref[0] = y1
    part = jnp.stack([jnp.sum(y1, axis=1), jnp.sum(y1 * y1, axis=1)], axis=0)

    @pl.when(jnp.logical_and(pl.program_id(0) == 0, t == 0))
    def _():
        st_ref[...] = jnp.zeros_like(st_ref)

    st_ref[...] += part


def _k3(tab, idx_r, W1):
    C = tab.shape[1]
    body = functools.partial(_k3_body, C=C)
    return pl.pallas_call(
        body,
        grid=(B, N // TILE_N),
        in_specs=[
            pl.BlockSpec((1, C, N), lambda b, t: (b, 0, 0)),
            pl.BlockSpec((1, 1, S_TILE), lambda b, t: (b * (N // TILE_N) + t, 0, 0)),
            pl.BlockSpec((64, 2 * C), lambda b, t: (0, 0)),
        ],
        out_specs=[
            pl.BlockSpec((1, 64, S_TILE), lambda b, t: (b, 0, t)),
            pl.BlockSpec((2, 64), lambda b, t: (0, 0)),
        ],
        out_shape=[
            jax.ShapeDtypeStruct((B, 64, N * K), jnp.float32),
            jax.ShapeDtypeStruct((2, 64), jnp.float32),
        ],
    )(tab, idx_r, W1)


def _k4_body(y1_ref, st_ref, g1_ref, b1_ref, w2_ref, a1_ref, st2_ref):
    s1 = st_ref[0]
    s2 = st_ref[1]
    mean = s1 / CNT
    var = s2 / CNT - mean * mean
    sc = g1_ref[0] / jnp.sqrt(var + EPS)
    bi = b1_ref[0] - sc * mean
    a1 = _leaky(sc[:, None] * y1_ref[0] + bi[:, None])
    a1_ref[0] = a1
    y2 = lax.dot_general(w2_ref[...], a1, (((1,), (0,)), ((), ())))
    part = jnp.stack([jnp.sum(y2, axis=1), jnp.sum(y2 * y2, axis=1)], axis=0)

    @pl.when(jnp.logical_and(pl.program_id(0) == 0, pl.program_id(1) == 0))
    def _():
        st2_ref[...] = jnp.zeros_like(st2_ref)

    st2_ref[...] += part


def _k4(y1, stats1, g1, b1, W2):
    return pl.pallas_call(
        _k4_body,
        grid=(B, N // TILE_N),
        in_specs=[
            pl.BlockSpec((1, 64, S_TILE), lambda b, t: (b, 0, t)),
            pl.BlockSpec((2, 64), lambda b, t: (0, 0)),
            pl.BlockSpec((1, 64), lambda b, t: (0, 0)),
            pl.BlockSpec((1, 64), lambda b, t: (0, 0)),
            pl.BlockSpec((64, 64), lambda b, t: (0, 0)),
        ],
        out_specs=[
            pl.BlockSpec((1, 64, S_TILE), lambda b, t: (b, 0, t)),
            pl.BlockSpec((2, 64), lambda b, t: (0, 0)),
        ],
        out_shape=[
            jax.ShapeDtypeStruct((B, 64, N * K), jnp.float32),
            jax.ShapeDtypeStruct((2, 64), jnp.float32),
        ],
    )(y1, stats1, g1.reshape(1, 64), b1.reshape(1, 64), W2)


def _k5_body(a1_ref, st2_ref, w2_ref, g2_ref, b2_ref, o_ref):
    W2 = w2_ref[...]
    mean = st2_ref[0] / CNT
    var = st2_ref[1] / CNT - mean * mean
    sc = g2_ref[0] / jnp.sqrt(var + EPS)
    bi = b2_ref[0] - sc * mean
    sgn = jnp.where(sc >= 0, 1.0, -1.0)
    W2s = sgn[:, None] * W2
    y2 = lax.dot_general(W2s, a1_ref[0], (((1,), (0,)), ((), ())))
    zh = jnp.max(y2.reshape(64, TILE_N, K), axis=-1)
    z = sgn[:, None] * zh
    o_ref[0] = _leaky(sc[:, None] * z + bi[:, None])


def _k5(a1, stats2, W2, g2, b2):
    return pl.pallas_call(
        _k5_body,
        grid=(B, N // TILE_N),
        in_specs=[
            pl.BlockSpec((1, 64, S_TILE), lambda b, t: (b, 0, t)),
            pl.BlockSpec((2, 64), lambda b, t: (0, 0)),
            pl.BlockSpec((64, 64), lambda b, t: (0, 0)),
            pl.BlockSpec((1, 64), lambda b, t: (0, 0)),
            pl.BlockSpec((1, 64), lambda b, t: (0, 0)),
        ],
        out_specs=pl.BlockSpec((1, 64, TILE_N), lambda b, t: (b, 0, t)),
        out_shape=jax.ShapeDtypeStruct((B, 64, N), jnp.float32),
    )(a1, stats2, W2, g2.reshape(1, 64), b2.reshape(1, 64))


ROWS_BLK = 8
NWORK = 32
ROWS_PER_W = (B * N) // NWORK
SV = 1024 + 32


def _rev(x):
    return lax.rev(x, (0,))


def _sortkv(v, i):
    return plsc.sort_key_val(v, i, descending=True)


def _merge16v(a, b):
    rb = _rev(b)
    u = jnp.maximum(a, rb)
    l = jnp.minimum(a, rb)
    iw = lax.iota(jnp.int32, 16)
    u, _ = _sortkv(u, iw)
    l, _ = _sortkv(l, iw)
    return u, l


def _merge16kv(av, ai, bv, bi):
    rbv, rbi = _rev(bv), _rev(bi)
    m = av >= rbv
    uv = jnp.where(m, av, rbv)
    ui = jnp.where(m, ai, rbi)
    lv = jnp.where(m, rbv, av)
    li = jnp.where(m, rbi, ai)
    uv, ui = _sortkv(uv, ui)
    lv, li = _sortkv(lv, li)
    return uv, ui, lv, li


def _merge32_16(thi, tih, tlo, til, cv, ci):
    rcv, rci = _rev(cv), _rev(ci)
    m = tlo >= rcv
    mv = jnp.where(m, tlo, rcv)
    mi = jnp.where(m, til, rci)
    m2 = thi >= mv
    av = jnp.where(m2, thi, mv)
    ai = jnp.where(m2, tih, mi)
    bv = jnp.where(m2, mv, thi)
    bi = jnp.where(m2, mi, tih)
    av, ai = _sortkv(av, ai)
    bv, bi = _sortkv(bv, bi)
    return av, ai, bv, bi


def _sc_topk_body(nd_ref, idx_ref, row_buf, out_buf, sv_val, sv_idx):
    wid = lax.axis_index("s") * 2 + lax.axis_index("c")
    row0 = wid * ROWS_PER_W
    iota16 = lax.iota(jnp.int32, 16)
    neg = jnp.full((16,), -jnp.inf, jnp.float32)

    def row_body(r, _):

        def chunk(j):
            return row_buf[r, pl.ds(j * 16, 16)]

        gm = []
        for gi in range(4):
            m = chunk(gi * 16)
            for j in range(1, 16):
                m = jnp.maximum(m, chunk(gi * 16 + j))
            gm.append(_sortkv(m, iota16)[0])
        u1, l1 = _merge16v(gm[0], gm[1])
        u2, l2 = _merge16v(gm[2], gm[3])
        p1 = jnp.maximum(u1, _rev(l2))
        p2 = jnp.maximum(l1, _rev(u2))
        t0 = jnp.min(jnp.minimum(p1, p2))

        cnts = []
        for j in range(64):
            cnts.append(jnp.sum((chunk(j) >= t0).astype(jnp.int32)))
        offs = [jnp.int32(0)]
        for j in range(64):
            offs.append(offs[-1] + cnts[j])
        total = offs[64]

        for j in range(9):
            sv_val[pl.ds(j * 16, 16)] = neg

        @pl.when(total > 128)
        def _():
            for j in range(9, SV // 16):
                sv_val[pl.ds(j * 16, 16)] = neg

        for j in range(64):
            c = chunk(j)
            m = c >= t0
            pos = offs[j] + plsc.cumsum(m.astype(jnp.int32)) - 1
            plsc.store_scatter(sv_val, [pos], c, mask=m)
            plsc.store_scatter(sv_idx, [pos], iota16 + j * 16, mask=m)

        ngrp = (total + 15) // 16
        s0v, s0i = _sortkv(sv_val[pl.ds(0, 16)], sv_idx[pl.ds(0, 16)])
        s1v, s1i = _sortkv(sv_val[pl.ds(16, 16)], sv_idx[pl.ds(16, 16)])
        thi, tih, tlo, til = _merge16kv(s0v, s0i, s1v, s1i)

        def wcond(st):
            return st[0] < ngrp

        def wbody(st):
            g, thi, tih, tlo, til = st
            cv = sv_val[pl.ds(g * 16, 16)]
            ci = sv_idx[pl.ds(g * 16, 16)]
            cv, ci = _sortkv(cv, ci)
            thi, tih, tlo, til = _merge32_16(thi, tih, tlo, til, cv, ci)
            return (g + 1, thi, tih, tlo, til)

        st = lax.while_loop(wcond, wbody, (jnp.int32(2), thi, tih, tlo, til))
        _, thi, tih, tlo, til = st
        out_buf[r, pl.ds(0, 16)] = tih
        out_buf[r, pl.ds(16, 16)] = til
        return 0

    def blk_body(blk, _):
        r0 = row0 + blk * ROWS_BLK
        pltpu.sync_copy(nd_ref.at[pl.ds(r0, ROWS_BLK)], row_buf)
        lax.fori_loop(0, ROWS_BLK, row_body, 0)
        pltpu.sync_copy(out_buf, idx_ref.at[pl.ds(r0, ROWS_BLK)])
        return 0

    lax.fori_loop(0, ROWS_PER_W // ROWS_BLK, blk_body, 0)


@jax.jit
def _sc_topk(nd2d):
    mesh = plsc.VectorSubcoreMesh(core_axis_name="c", subcore_axis_name="s")
    f = functools.partial(
        pl.kernel,
        out_type=jax.ShapeDtypeStruct((B * N, K), jnp.int32),
        mesh=mesh,
        compiler_params=pltpu.CompilerParams(needs_layout_passes=False),
        scratch_types=[
            pltpu.VMEM((ROWS_BLK, N), jnp.float32),
            pltpu.VMEM((ROWS_BLK, K), jnp.int32),
            pltpu.VMEM((SV,), jnp.float32),
            pltpu.VMEM((SV,), jnp.int32),
        ],
    )(_sc_topk_body)
    return f(nd2d)


def _topk_idx(nd):
    idx2d = _sc_topk(nd.reshape(B * N, N))
    return idx2d.reshape(B, N, K)


def _edge_layer(tab, W1, g1, b1, W2, g2, b2):
    nd = _neg_dist(tab)
    idx = _topk_idx(nd)
    idx_r = idx.reshape(B * (N // TILE_N), 1, S_TILE)
    y1, stats1 = _k3(tab, idx_r, W1)
    a1, stats2 = _k4(y1, stats1, g1, b1, W2)
    return _k5(a1, stats2, W2, g2, b2)


def kernel(x, W1_0, g1_0, b1_0, W2_0, g2_0, b2_0, W1_1, g1_1, b1_1, W2_1, g2_1, b2_1):
    h = _edge_layer(x, W1_0, g1_0, b1_0, W2_0, g2_0, b2_0)
    h = _edge_layer(h, W1_1, g1_1, b1_1, W2_1, g2_1, b2_1)
    return h


# speedup vs baseline: 7.0567x; 1.4724x over previous
import functools
import jax
import jax.numpy as jnp
from jax import lax
from jax.experimental import pallas as pl
from jax.experimental.pallas import tpu as pltpu
from jax.experimental.pallas import tpu_sc as plsc

EPS = 1e-5
B = 8
N = 1024
K = 32
TILE_N = 256
S_TILE = TILE_N * K
CNT = B * N * K


def _leaky(y):
    return jnp.where(y >= 0, y, 0.2 * y)


def _nd_body(tab_ref, o_ref):
    tb = tab_ref[0]
    G = lax.dot_general(tb, tb, (((0,), (0,)), ((), ())))
    xx = jnp.sum(tb * tb, axis=0)
    inner = -2.0 * G
    o_ref[0] = (-xx[:, None] - inner) - xx[None, :]


def _neg_dist(tab):
    C = tab.shape[1]
    return pl.pallas_call(
        _nd_body,
        grid=(B,),
        in_specs=[pl.BlockSpec((1, C, N), lambda b: (b, 0, 0))],
        out_specs=pl.BlockSpec((1, N, N), lambda b: (b, 0, 0)),
        out_shape=jax.ShapeDtypeStruct((B, N, N), jnp.float32),
    )(tab)


def _gather_lanes(tab, idxb):
    C, S = idxb.shape
    wid = lax.shift_right_logical(idxb, 7)
    idx_lo = jnp.bitwise_and(idxb, 127)
    out = jnp.zeros((C, S), jnp.float32)
    for w in range(N // 128):
        g = jnp.take_along_axis(tab[:, w * 128:(w + 1) * 128], idx_lo, axis=1)
        out = jnp.where(wid == w, g, out)
    return out


def _k3_body(tab_ref, idx_ref, w1_ref, y1_ref, st_ref, *, C):
    tb = tab_ref[0]
    t = pl.program_id(1)
    base = t * TILE_N
    idx = idx_ref[0, 0]
    idxb = jnp.broadcast_to(idx[None, :], (C, S_TILE))
    feat = _gather_lanes(tb, idxb)
    center = jnp.tile(tab_ref[0, :, pl.ds(base, TILE_N)], (1, K))
    g = jnp.concatenate([center, feat - center], axis=0)
    y1 = lax.dot_general(w1_ref[...], g, (((1,), (0,)), ((), ())))
    y1_ref[0] = y1
    part = jnp.stack([jnp.sum(y1, axis=1), jnp.sum(y1 * y1, axis=1)], axis=0)

    @pl.when(jnp.logical_and(pl.program_id(0) == 0, t == 0))
    def _():
        st_ref[...] = jnp.zeros_like(st_ref)

    st_ref[...] += part


def _k3(tab, idx_r, W1):
    C = tab.shape[1]
    body = functools.partial(_k3_body, C=C)
    return pl.pallas_call(
        body,
        grid=(B, N // TILE_N),
        in_specs=[
            pl.BlockSpec((1, C, N), lambda b, t: (b, 0, 0)),
            pl.BlockSpec((1, 1, S_TILE), lambda b, t: (b * (N // TILE_N) + t, 0, 0)),
            pl.BlockSpec((64, 2 * C), lambda b, t: (0, 0)),
        ],
        out_specs=[
            pl.BlockSpec((1, 64, S_TILE), lambda b, t: (b, 0, t)),
            pl.BlockSpec((2, 64), lambda b, t: (0, 0)),
        ],
        out_shape=[
            jax.ShapeDtypeStruct((B, 64, N * K), jnp.float32),
            jax.ShapeDtypeStruct((2, 64), jnp.float32),
        ],
    )(tab, idx_r, W1)


def _k4_body(y1_ref, st_ref, g1_ref, b1_ref, w2_ref, a1_ref, st2_ref):
    s1 = st_ref[0]
    s2 = st_ref[1]
    mean = s1 / CNT
    var = s2 / CNT - mean * mean
    sc = g1_ref[0] / jnp.sqrt(var + EPS)
    bi = b1_ref[0] - sc * mean
    a1 = _leaky(sc[:, None] * y1_ref[0] + bi[:, None])
    a1_ref[0] = a1
    y2 = lax.dot_general(w2_ref[...], a1, (((1,), (0,)), ((), ())))
    part = jnp.stack([jnp.sum(y2, axis=1), jnp.sum(y2 * y2, axis=1)], axis=0)

    @pl.when(jnp.logical_and(pl.program_id(0) == 0, pl.program_id(1) == 0))
    def _():
        st2_ref[...] = jnp.zeros_like(st2_ref)

    st2_ref[...] += part


def _k4(y1, stats1, g1, b1, W2):
    return pl.pallas_call(
        _k4_body,
        grid=(B, N // TILE_N),
        in_specs=[
            pl.BlockSpec((1, 64, S_TILE), lambda b, t: (b, 0, t)),
            pl.BlockSpec((2, 64), lambda b, t: (0, 0)),
            pl.BlockSpec((1, 64), lambda b, t: (0, 0)),
            pl.BlockSpec((1, 64), lambda b, t: (0, 0)),
            pl.BlockSpec((64, 64), lambda b, t: (0, 0)),
        ],
        out_specs=[
            pl.BlockSpec((1, 64, S_TILE), lambda b, t: (b, 0, t)),
            pl.BlockSpec((2, 64), lambda b, t: (0, 0)),
        ],
        out_shape=[
            jax.ShapeDtypeStruct((B, 64, N * K), jnp.float32),
            jax.ShapeDtypeStruct((2, 64), jnp.float32),
        ],
    )(y1, stats1, g1.reshape(1, 64), b1.reshape(1, 64), W2)


def _k5_body(a1_ref, st2_ref, w2_ref, g2_ref, b2_ref, o_ref):
    W2 = w2_ref[...]
    mean = st2_ref[0] / CNT
    var = st2_ref[1] / CNT - mean * mean
    sc = g2_ref[0] / jnp.sqrt(var + EPS)
    bi = b2_ref[0] - sc * mean
    sgn = jnp.where(sc >= 0, 1.0, -1.0)
    W2s = sgn[:, None] * W2
    y2 = lax.dot_general(W2s, a1_ref[0], (((1,), (0,)), ((), ())))
    zh = jnp.max(y2.reshape(64, K, TILE_N), axis=1)
    z = sgn[:, None] * zh
    o_ref[0] = _leaky(sc[:, None] * z + bi[:, None])


def _k5(a1, stats2, W2, g2, b2):
    return pl.pallas_call(
        _k5_body,
        grid=(B, N // TILE_N),
        in_specs=[
            pl.BlockSpec((1, 64, S_TILE), lambda b, t: (b, 0, t)),
            pl.BlockSpec((2, 64), lambda b, t: (0, 0)),
            pl.BlockSpec((64, 64), lambda b, t: (0, 0)),
            pl.BlockSpec((1, 64), lambda b, t: (0, 0)),
            pl.BlockSpec((1, 64), lambda b, t: (0, 0)),
        ],
        out_specs=pl.BlockSpec((1, 64, TILE_N), lambda b, t: (b, 0, t)),
        out_shape=jax.ShapeDtypeStruct((B, 64, N), jnp.float32),
    )(a1, stats2, W2, g2.reshape(1, 64), b2.reshape(1, 64))


ROWS_BLK = 8
NWORK = 32
ROWS_PER_W = (B * N) // NWORK
SV = 1024 + 32


def _rev(x):
    return lax.rev(x, (0,))


def _sortkv(v, i):
    return plsc.sort_key_val(v, i, descending=True)


def _merge16v(a, b):
    rb = _rev(b)
    u = jnp.maximum(a, rb)
    l = jnp.minimum(a, rb)
    iw = lax.iota(jnp.int32, 16)
    u, _ = _sortkv(u, iw)
    l, _ = _sortkv(l, iw)
    return u, l


def _merge16kv(av, ai, bv, bi):
    rbv, rbi = _rev(bv), _rev(bi)
    m = av >= rbv
    uv = jnp.where(m, av, rbv)
    ui = jnp.where(m, ai, rbi)
    lv = jnp.where(m, rbv, av)
    li = jnp.where(m, rbi, ai)
    uv, ui = _sortkv(uv, ui)
    lv, li = _sortkv(lv, li)
    return uv, ui, lv, li


def _merge32_16(thi, tih, tlo, til, cv, ci):
    rcv, rci = _rev(cv), _rev(ci)
    m = tlo >= rcv
    mv = jnp.where(m, tlo, rcv)
    mi = jnp.where(m, til, rci)
    m2 = thi >= mv
    av = jnp.where(m2, thi, mv)
    ai = jnp.where(m2, tih, mi)
    bv = jnp.where(m2, mv, thi)
    bi = jnp.where(m2, mi, tih)
    av, ai = _sortkv(av, ai)
    bv, bi = _sortkv(bv, bi)
    return av, ai, bv, bi


def _sc_topk_body(nd_ref, idx_ref, row_buf, out_buf, sv_val, sv_idx):
    wid = lax.axis_index("s") * 2 + lax.axis_index("c")
    row0 = wid * ROWS_PER_W
    iota16 = lax.iota(jnp.int32, 16)
    neg = jnp.full((16,), -jnp.inf, jnp.float32)

    def row_body(r, _):

        def chunk(j):
            return row_buf[r, pl.ds(j * 16, 16)]

        gm = []
        for gi in range(4):
            m = chunk(gi * 16)
            for j in range(1, 16):
                m = jnp.maximum(m, chunk(gi * 16 + j))
            gm.append(_sortkv(m, iota16)[0])
        u1, l1 = _merge16v(gm[0], gm[1])
        u2, l2 = _merge16v(gm[2], gm[3])
        p1 = jnp.maximum(u1, _rev(l2))
        p2 = jnp.maximum(l1, _rev(u2))
        t0 = jnp.min(jnp.minimum(p1, p2))

        cnts = []
        for j in range(64):
            cnts.append(jnp.sum((chunk(j) >= t0).astype(jnp.int32)))
        offs = [jnp.int32(0)]
        for j in range(64):
            offs.append(offs[-1] + cnts[j])
        total = offs[64]

        for j in range(9):
            sv_val[pl.ds(j * 16, 16)] = neg

        @pl.when(total > 128)
        def _():
            for j in range(9, SV // 16):
                sv_val[pl.ds(j * 16, 16)] = neg

        for j in range(64):
            c = chunk(j)
            m = c >= t0
            pos = offs[j] + plsc.cumsum(m.astype(jnp.int32)) - 1
            plsc.store_scatter(sv_val, [pos], c, mask=m)
            plsc.store_scatter(sv_idx, [pos], iota16 + j * 16, mask=m)

        ngrp = (total + 15) // 16
        s0v, s0i = _sortkv(sv_val[pl.ds(0, 16)], sv_idx[pl.ds(0, 16)])
        s1v, s1i = _sortkv(sv_val[pl.ds(16, 16)], sv_idx[pl.ds(16, 16)])
        thi, tih, tlo, til = _merge16kv(s0v, s0i, s1v, s1i)

        def wcond(st):
            return st[0] < ngrp

        def wbody(st):
            g, thi, tih, tlo, til = st
            cv = sv_val[pl.ds(g * 16, 16)]
            ci = sv_idx[pl.ds(g * 16, 16)]
            cv, ci = _sortkv(cv, ci)
            thi, tih, tlo, til = _merge32_16(thi, tih, tlo, til, cv, ci)
            return (g + 1, thi, tih, tlo, til)

        st = lax.while_loop(wcond, wbody, (jnp.int32(2), thi, tih, tlo, til))
        _, thi, tih, tlo, til = st
        out_buf[r, pl.ds(0, 16)] = tih
        out_buf[r, pl.ds(16, 16)] = til
        return 0

    def blk_body(blk, _):
        r0 = row0 + blk * ROWS_BLK
        pltpu.sync_copy(nd_ref.at[pl.ds(r0, ROWS_BLK)], row_buf)
        lax.fori_loop(0, ROWS_BLK, row_body, 0)
        pltpu.sync_copy(out_buf, idx_ref.at[pl.ds(r0, ROWS_BLK)])
        return 0

    lax.fori_loop(0, ROWS_PER_W // ROWS_BLK, blk_body, 0)


@jax.jit
def _sc_topk(nd2d):
    mesh = plsc.VectorSubcoreMesh(core_axis_name="c", subcore_axis_name="s")
    f = functools.partial(
        pl.kernel,
        out_type=jax.ShapeDtypeStruct((B * N, K), jnp.int32),
        mesh=mesh,
        compiler_params=pltpu.CompilerParams(needs_layout_passes=False),
        scratch_types=[
            pltpu.VMEM((ROWS_BLK, N), jnp.float32),
            pltpu.VMEM((ROWS_BLK, K), jnp.int32),
            pltpu.VMEM((SV,), jnp.float32),
            pltpu.VMEM((SV,), jnp.int32),
        ],
    )(_sc_topk_body)
    return f(nd2d)


def _topk_idx(nd):
    idx2d = _sc_topk(nd.reshape(B * N, N))
    return idx2d.reshape(B, N, K)


def _edge_layer(tab, W1, g1, b1, W2, g2, b2):
    nd = _neg_dist(tab)
    idx = _topk_idx(nd)
    idx_r = idx.transpose(0, 2, 1).reshape(B, K, N // TILE_N, TILE_N)
    idx_r = idx_r.transpose(0, 2, 1, 3).reshape(B * (N // TILE_N), 1, S_TILE)
    y1, stats1 = _k3(tab, idx_r, W1)
    a1, stats2 = _k4(y1, stats1, g1, b1, W2)
    return _k5(a1, stats2, W2, g2, b2)


def kernel(x, W1_0, g1_0, b1_0, W2_0, g2_0, b2_0, W1_1, g1_1, b1_1, W2_1, g2_1, b2_1):
    h = _edge_layer(x, W1_0, g1_0, b1_0, W2_0, g2_0, b2_0)
    h = _edge_layer(h, W1_1, g1_1, b1_1, W2_1, g2_1, b2_1)
    return h


# speedup vs baseline: 8.2246x; 1.1655x over previous
import functools
import jax
import jax.numpy as jnp
from jax import lax
from jax.experimental import pallas as pl
from jax.experimental.pallas import tpu as pltpu
from jax.experimental.pallas import tpu_sc as plsc

EPS = 1e-5
B = 8
N = 1024
K = 32
TILE_N = 256
S_TILE = TILE_N * K
CNT = B * N * K


def _leaky(y):
    return jnp.where(y >= 0, y, 0.2 * y)


def _nd_body(tab_ref, o_ref):
    tb = tab_ref[0]
    G = lax.dot_general(tb, tb, (((0,), (0,)), ((), ())))
    xx = jnp.sum(tb * tb, axis=0)
    inner = -2.0 * G
    o_ref[0] = (-xx[:, None] - inner) - xx[None, :]


def _neg_dist(tab):
    C = tab.shape[1]
    return pl.pallas_call(
        _nd_body,
        grid=(B,),
        in_specs=[pl.BlockSpec((1, C, N), lambda b: (b, 0, 0))],
        out_specs=pl.BlockSpec((1, N, N), lambda b: (b, 0, 0)),
        out_shape=jax.ShapeDtypeStruct((B, N, N), jnp.float32),
    )(tab)


def _gather_lanes(tab, idxb):
    C, S = idxb.shape
    wid = lax.shift_right_logical(idxb, 7)
    idx_lo = jnp.bitwise_and(idxb, 127)
    out = jnp.zeros((C, S), jnp.float32)
    for w in range(N // 128):
        g = jnp.take_along_axis(tab[:, w * 128:(w + 1) * 128], idx_lo, axis=1)
        out = jnp.where(wid == w, g, out)
    return out


def _k3_body(tab_ref, idx_ref, w1_ref, y1_ref, st_ref, *, C):
    tb = tab_ref[0]
    t = pl.program_id(1)
    base = t * TILE_N
    idx = idx_ref[0, 0]
    idxb = jnp.broadcast_to(idx[None, :], (C, S_TILE))
    feat = _gather_lanes(tb, idxb)
    center = jnp.tile(tab_ref[0, :, pl.ds(base, TILE_N)], (1, K))
    g = jnp.concatenate([center, feat - center], axis=0)
    y1 = lax.dot_general(w1_ref[...], g, (((1,), (0,)), ((), ())))
    y1_ref[0] = y1
    part = jnp.stack([jnp.sum(y1, axis=1), jnp.sum(y1 * y1, axis=1)], axis=0)

    @pl.when(jnp.logical_and(pl.program_id(0) == 0, t == 0))
    def _():
        st_ref[...] = jnp.zeros_like(st_ref)

    st_ref[...] += part


def _k3(tab, idx_r, W1):
    C = tab.shape[1]
    body = functools.partial(_k3_body, C=C)
    return pl.pallas_call(
        body,
        grid=(B, N // TILE_N),
        in_specs=[
            pl.BlockSpec((1, C, N), lambda b, t: (b, 0, 0)),
            pl.BlockSpec((1, 1, S_TILE), lambda b, t: (b * (N // TILE_N) + t, 0, 0)),
            pl.BlockSpec((64, 2 * C), lambda b, t: (0, 0)),
        ],
        out_specs=[
            pl.BlockSpec((1, 64, S_TILE), lambda b, t: (b, 0, t)),
            pl.BlockSpec((2, 64), lambda b, t: (0, 0)),
        ],
        out_shape=[
            jax.ShapeDtypeStruct((B, 64, N * K), jnp.float32),
            jax.ShapeDtypeStruct((2, 64), jnp.float32),
        ],
    )(tab, idx_r, W1)


def _k4_body(y1_ref, st_ref, g1_ref, b1_ref, w2_ref, a1_ref, st2_ref):
    s1 = st_ref[0]
    s2 = st_ref[1]
    mean = s1 / CNT
    var = s2 / CNT - mean * mean
    sc = g1_ref[0] / jnp.sqrt(var + EPS)
    bi = b1_ref[0] - sc * mean
    a1 = _leaky(sc[:, None] * y1_ref[0] + bi[:, None])
    a1_ref[0] = a1
    y2 = lax.dot_general(w2_ref[...], a1, (((1,), (0,)), ((), ())))
    part = jnp.stack([jnp.sum(y2, axis=1), jnp.sum(y2 * y2, axis=1)], axis=0)

    @pl.when(jnp.logical_and(pl.program_id(0) == 0, pl.program_id(1) == 0))
    def _():
        st2_ref[...] = jnp.zeros_like(st2_ref)

    st2_ref[...] += part


def _k4(y1, stats1, g1, b1, W2):
    return pl.pallas_call(
        _k4_body,
        grid=(B, N // TILE_N),
        in_specs=[
            pl.BlockSpec((1, 64, S_TILE), lambda b, t: (b, 0, t)),
            pl.BlockSpec((2, 64), lambda b, t: (0, 0)),
            pl.BlockSpec((1, 64), lambda b, t: (0, 0)),
            pl.BlockSpec((1, 64), lambda b, t: (0, 0)),
            pl.BlockSpec((64, 64), lambda b, t: (0, 0)),
        ],
        out_specs=[
            pl.BlockSpec((1, 64, S_TILE), lambda b, t: (b, 0, t)),
            pl.BlockSpec((2, 64), lambda b, t: (0, 0)),
        ],
        out_shape=[
            jax.ShapeDtypeStruct((B, 64, N * K), jnp.float32),
            jax.ShapeDtypeStruct((2, 64), jnp.float32),
        ],
    )(y1, stats1, g1.reshape(1, 64), b1.reshape(1, 64), W2)


def _k5_body(a1_ref, st2_ref, w2_ref, g2_ref, b2_ref, o_ref):
    W2 = w2_ref[...]
    mean = st2_ref[0] / CNT
    var = st2_ref[1] / CNT - mean * mean
    sc = g2_ref[0] / jnp.sqrt(var + EPS)
    bi = b2_ref[0] - sc * mean
    sgn = jnp.where(sc >= 0, 1.0, -1.0)
    W2s = sgn[:, None] * W2
    y2 = lax.dot_general(W2s, a1_ref[0], (((1,), (0,)), ((), ())))
    zh = jnp.max(y2.reshape(64, K, TILE_N), axis=1)
    z = sgn[:, None] * zh
    o_ref[0] = _leaky(sc[:, None] * z + bi[:, None])


def _k5(a1, stats2, W2, g2, b2):
    return pl.pallas_call(
        _k5_body,
        grid=(B, N // TILE_N),
        in_specs=[
            pl.BlockSpec((1, 64, S_TILE), lambda b, t: (b, 0, t)),
            pl.BlockSpec((2, 64), lambda b, t: (0, 0)),
            pl.BlockSpec((64, 64), lambda b, t: (0, 0)),
            pl.BlockSpec((1, 64), lambda b, t: (0, 0)),
            pl.BlockSpec((1, 64), lambda b, t: (0, 0)),
        ],
        out_specs=pl.BlockSpec((1, 64, TILE_N), lambda b, t: (b, 0, t)),
        out_shape=jax.ShapeDtypeStruct((B, 64, N), jnp.float32),
    )(a1, stats2, W2, g2.reshape(1, 64), b2.reshape(1, 64))


ROWS_BLK = 8
NWORK = 32
ROWS_PER_W = (B * N) // NWORK
SV = 1024 + 32


def _rev(x):
    return lax.rev(x, (0,))


def _sortkv(v, i):
    return plsc.sort_key_val(v, i, descending=True)


def _merge16v(a, b):
    rb = _rev(b)
    u = jnp.maximum(a, rb)
    l = jnp.minimum(a, rb)
    iw = lax.iota(jnp.int32, 16)
    u, _ = _sortkv(u, iw)
    l, _ = _sortkv(l, iw)
    return u, l


def _merge16kv(av, ai, bv, bi):
    rbv, rbi = _rev(bv), _rev(bi)
    m = av >= rbv
    uv = jnp.where(m, av, rbv)
    ui = jnp.where(m, ai, rbi)
    lv = jnp.where(m, rbv, av)
    li = jnp.where(m, rbi, ai)
    uv, ui = _sortkv(uv, ui)
    lv, li = _sortkv(lv, li)
    return uv, ui, lv, li


def _merge32_16(thi, tih, tlo, til, cv, ci):
    rcv, rci = _rev(cv), _rev(ci)
    m = tlo >= rcv
    mv = jnp.where(m, tlo, rcv)
    mi = jnp.where(m, til, rci)
    m2 = thi >= mv
    av = jnp.where(m2, thi, mv)
    ai = jnp.where(m2, tih, mi)
    bv = jnp.where(m2, mv, thi)
    bi = jnp.where(m2, mi, tih)
    av, ai = _sortkv(av, ai)
    bv, bi = _sortkv(bv, bi)
    return av, ai, bv, bi


def _sc_topk_body(nd_ref, idx_ref, row_buf, out_buf, sv_val, sv_idx):
    wid = lax.axis_index("s") * 2 + lax.axis_index("c")
    row0 = wid * ROWS_PER_W
    iota16 = lax.iota(jnp.int32, 16)
    neg = jnp.full((16,), -jnp.inf, jnp.float32)

    def row_body(r, _):

        def chunk(j):
            return row_buf[r, pl.ds(j * 16, 16)]

        gm = []
        for gi in range(4):
            m = chunk(gi * 16)
            for j in range(1, 16):
                m = jnp.maximum(m, chunk(gi * 16 + j))
            gm.append(_sortkv(m, iota16)[0])
        u1, l1 = _merge16v(gm[0], gm[1])
        u2, l2 = _merge16v(gm[2], gm[3])
        p1 = jnp.maximum(u1, _rev(l2))
        p2 = jnp.maximum(l1, _rev(u2))
        t0 = jnp.min(jnp.minimum(p1, p2))

        off = jnp.int32(0)
        for j in range(64):
            c = chunk(j)
            m = c >= t0
            pc = plsc.all_reduce_population_count(m)
            plsc.store_compressed(sv_val.at[pl.ds(off, 16)], c, mask=m)
            plsc.store_compressed(sv_idx.at[pl.ds(off, 16)], iota16 + j * 16, mask=m)
            off = off + jnp.squeeze(lax.slice(pc, (0,), (1,)))
        total = off
        plsc.store_scatter(sv_val, [total + iota16], neg)

        ngrp = (total + 15) // 16
        s0v, s0i = _sortkv(sv_val[pl.ds(0, 16)], sv_idx[pl.ds(0, 16)])
        s1v, s1i = _sortkv(sv_val[pl.ds(16, 16)], sv_idx[pl.ds(16, 16)])
        thi, tih, tlo, til = _merge16kv(s0v, s0i, s1v, s1i)

        def wcond(st):
            return st[0] < ngrp

        def wbody(st):
            g, thi, tih, tlo, til = st
            cv = sv_val[pl.ds(g * 16, 16)]
            ci = sv_idx[pl.ds(g * 16, 16)]
            cv, ci = _sortkv(cv, ci)
            thi, tih, tlo, til = _merge32_16(thi, tih, tlo, til, cv, ci)
            return (g + 1, thi, tih, tlo, til)

        st = lax.while_loop(wcond, wbody, (jnp.int32(2), thi, tih, tlo, til))
        _, thi, tih, tlo, til = st
        out_buf[r, pl.ds(0, 16)] = tih
        out_buf[r, pl.ds(16, 16)] = til
        return 0

    def blk_body(blk, _):
        r0 = row0 + blk * ROWS_BLK
        pltpu.sync_copy(nd_ref.at[pl.ds(r0, ROWS_BLK)], row_buf)
        lax.fori_loop(0, ROWS_BLK, row_body, 0)
        pltpu.sync_copy(out_buf, idx_ref.at[pl.ds(r0, ROWS_BLK)])
        return 0

    lax.fori_loop(0, ROWS_PER_W // ROWS_BLK, blk_body, 0)


@jax.jit
def _sc_topk(nd2d):
    mesh = plsc.VectorSubcoreMesh(core_axis_name="c", subcore_axis_name="s")
    f = functools.partial(
        pl.kernel,
        out_type=jax.ShapeDtypeStruct((B * N, K), jnp.int32),
        mesh=mesh,
        compiler_params=pltpu.CompilerParams(needs_layout_passes=False),
        scratch_types=[
            pltpu.VMEM((ROWS_BLK, N), jnp.float32),
            pltpu.VMEM((ROWS_BLK, K), jnp.int32),
            pltpu.VMEM((SV,), jnp.float32),
            pltpu.VMEM((SV,), jnp.int32),
        ],
    )(_sc_topk_body)
    return f(nd2d)


def _topk_idx(nd):
    idx2d = _sc_topk(nd.reshape(B * N, N))
    return idx2d.reshape(B, N, K)


def _edge_layer(tab, W1, g1, b1, W2, g2, b2):
    nd = _neg_dist(tab)
    idx = _topk_idx(nd)
    idx_r = idx.transpose(0, 2, 1).reshape(B, K, N // TILE_N, TILE_N)
    idx_r = idx_r.transpose(0, 2, 1, 3).reshape(B * (N // TILE_N), 1, S_TILE)
    y1, stats1 = _k3(tab, idx_r, W1)
    a1, stats2 = _k4(y1, stats1, g1, b1, W2)
    return _k5(a1, stats2, W2, g2, b2)


def kernel(x, W1_0, g1_0, b1_0, W2_0, g2_0, b2_0, W1_1, g1_1, b1_1, W2_1, g2_1, b2_1):
    h = _edge_layer(x, W1_0, g1_0, b1_0, W2_0, g2_0, b2_0)
    h = _edge_layer(h, W1_1, g1_1, b1_1, W2_1, g2_1, b2_1)
    return h
